# baseline (speedup 1.0000x reference)
import functools
import jax
import jax.numpy as jnp
from jax import lax
from jax.experimental import pallas as pl
from jax.experimental.pallas import tpu as pltpu
from jax.experimental.pallas import tpu_sc as plsc

D = 128
B, L = 4096, 200
ROWS = B * L
XR = ROWS // D
XBLK = 128
NC, NS = 2, 16
NW = NC * NS
RPW = ROWS // NW
CH = 128
NCHUNK = RPW // CH
TROWS = 32
NB = 4


def _idx_body(xp_ref, m1_ref, m2_ref, c_ref, t_ref):
    xp = xp_ref[...].reshape(XBLK, D, 2)
    c_ref[...] = xp[:, :, 0] * 4 + xp[:, :, 1]
    m2 = m2_ref[...].reshape(8, 1, D)
    m1 = m1_ref[...].reshape(1, 4, D)
    t_ref[...] = (jnp.broadcast_to(m2, (8, 4, D)) +
                  jnp.broadcast_to(m1, (8, 4, D))).reshape(TROWS, D)


_idx_kernel = pl.pallas_call(
    _idx_body,
    grid=(XR // XBLK,),
    in_specs=[
        pl.BlockSpec((XBLK, 2 * D), lambda i: (i, 0)),
        pl.BlockSpec((4, D), lambda i: (0, 0)),
        pl.BlockSpec((8, D), lambda i: (0, 0)),
    ],
    out_specs=[
        pl.BlockSpec((XBLK, D), lambda i: (i, 0)),
        pl.BlockSpec((TROWS, D), lambda i: (0, 0)),
    ],
    out_shape=[
        jax.ShapeDtypeStruct((XR, D), jnp.int32),
        jax.ShapeDtypeStruct((TROWS, D), jnp.float32),
    ],
)

_mesh = plsc.VectorSubcoreMesh(core_axis_name="c", subcore_axis_name="s")


@functools.partial(
    pl.kernel,
    out_type=jax.ShapeDtypeStruct((ROWS, D), jnp.float32),
    mesh=_mesh,
    scratch_types=[
        pltpu.VMEM((TROWS, D), jnp.float32),
        pltpu.VMEM_SHARED((TROWS, D), jnp.float32),
        pltpu.VMEM((NB, CH), jnp.int32),
        pltpu.VMEM((NB, CH, D), jnp.float32),
        [pltpu.SemaphoreType.DMA] * NB,
        [pltpu.SemaphoreType.DMA] * NB,
    ],
)
def _sc_embed(c_hbm, t_hbm, out_hbm, t_v, t_sh, idx_v, rows_v, gsems, osems):
    cid = lax.axis_index("c")
    sid = lax.axis_index("s")
    wid = cid * NS + sid
    w0 = wid * RPW

    @pl.when(sid == 0)
    def _stage():
        pltpu.sync_copy(t_hbm, t_v)
        pltpu.sync_copy(t_v, t_sh)

    plsc.subcore_barrier()

    def start_chunk(g, b):
        base = w0 + g * CH
        pltpu.sync_copy(c_hbm.at[pl.ds(base, CH)], idx_v.at[b])
        pltpu.async_copy(t_sh.at[idx_v.at[b]], rows_v.at[b], gsems[b])

    def finish_chunk(g, b):
        base = w0 + g * CH
        pltpu.make_async_copy(t_sh.at[idx_v.at[b]], rows_v.at[b], gsems[b]).wait()
        pltpu.async_copy(rows_v.at[b], out_hbm.at[pl.ds(base, CH)], osems[b])

    def drain_out(g, b):
        base = w0 + g * CH
        pltpu.make_async_copy(
            rows_v.at[b], out_hbm.at[pl.ds(base, CH)], osems[b]).wait()

    for b in range(NB):
        start_chunk(b, b)

    def super_body(gg, carry):
        g0 = gg * NB
        for b in range(NB):
            g = g0 + b
            finish_chunk(g, b)

            @pl.when(g + NB < NCHUNK)
            def _next():
                drain_out(g, b)
                start_chunk(g + NB, b)
        return carry

    lax.fori_loop(0, NCHUNK // NB, super_body, 0)

    for b in range(NB):
        drain_out(NCHUNK - NB + b, b)


def kernel(x, min1_w, min2_w):
    xp = x.astype(jnp.int32).reshape(XR, 2 * D)
    c, t = _idx_kernel(xp, min1_w, min2_w)
    out = _sc_embed(c.reshape(ROWS), t)
    return out.reshape(B, L, D)


# speedup vs baseline: 6.9373x; 6.9373x over previous
import functools
import jax
import jax.numpy as jnp
from jax import lax
from jax.experimental import pallas as pl
from jax.experimental.pallas import tpu as pltpu
from jax.experimental.pallas import tpu_sc as plsc

D = 128
B, L = 4096, 200
ROWS = B * L
XR = ROWS // D
XBLK = 128
NC, NS = 2, 16
NW = NC * NS
RPW = ROWS // NW
CH = 128
NCHUNK = RPW // CH
TROWS = 32
NB = 4


def _idx_body(x0_ref, x1_ref, m1_ref, m2_ref, c_ref, t_ref):
    c_ref[...] = x0_ref[...] * 4 + x1_ref[...]
    m2 = m2_ref[...].reshape(8, 1, D)
    m1 = m1_ref[...].reshape(1, 4, D)
    t_ref[...] = (jnp.broadcast_to(m2, (8, 4, D)) +
                  jnp.broadcast_to(m1, (8, 4, D))).reshape(TROWS, D)


_idx_kernel = pl.pallas_call(
    _idx_body,
    grid=(XR // XBLK,),
    in_specs=[
        pl.BlockSpec((XBLK, D), lambda i: (i, 0)),
        pl.BlockSpec((XBLK, D), lambda i: (i, 0)),
        pl.BlockSpec((4, D), lambda i: (0, 0)),
        pl.BlockSpec((8, D), lambda i: (0, 0)),
    ],
    out_specs=[
        pl.BlockSpec((XBLK, D), lambda i: (i, 0)),
        pl.BlockSpec((TROWS, D), lambda i: (0, 0)),
    ],
    out_shape=[
        jax.ShapeDtypeStruct((XR, D), jnp.int32),
        jax.ShapeDtypeStruct((TROWS, D), jnp.float32),
    ],
)

_mesh = plsc.VectorSubcoreMesh(core_axis_name="c", subcore_axis_name="s")


@functools.partial(
    pl.kernel,
    out_type=jax.ShapeDtypeStruct((ROWS, D), jnp.float32),
    mesh=_mesh,
    scratch_types=[
        pltpu.VMEM((TROWS, D), jnp.float32),
        pltpu.VMEM_SHARED((TROWS, D), jnp.float32),
        pltpu.VMEM((NCHUNK, CH), jnp.int32),
        pltpu.VMEM((NB, CH, D), jnp.float32),
        [pltpu.SemaphoreType.DMA] * NB,
        [pltpu.SemaphoreType.DMA] * NB,
    ],
)
def _sc_embed(c_hbm, t_hbm, out_hbm, t_v, t_sh, idx_v, rows_v, gsems, osems):
    cid = lax.axis_index("c")
    sid = lax.axis_index("s")
    wid = cid * NS + sid
    w0 = wid * RPW

    @pl.when(sid == 0)
    def _stage():
        pltpu.sync_copy(t_hbm, t_v)
        pltpu.sync_copy(t_v, t_sh)

    pltpu.sync_copy(c_hbm.at[pl.ds(wid * NCHUNK, NCHUNK)], idx_v)

    plsc.subcore_barrier()

    def start_chunk(g, b):
        pltpu.async_copy(t_sh.at[idx_v.at[g]], rows_v.at[b], gsems[b])

    def finish_chunk(g, b):
        base = w0 + g * CH
        pltpu.make_async_copy(t_sh.at[idx_v.at[g]], rows_v.at[b], gsems[b]).wait()
        pltpu.async_copy(rows_v.at[b], out_hbm.at[pl.ds(base, CH)], osems[b])

    def drain_out(g, b):
        base = w0 + g * CH
        pltpu.make_async_copy(
            rows_v.at[b], out_hbm.at[pl.ds(base, CH)], osems[b]).wait()

    for b in range(NB):
        start_chunk(b, b)

    def super_body(gg, carry):
        g0 = gg * NB
        for b in range(NB):
            g = g0 + b
            finish_chunk(g, b)

            @pl.when(g + NB < NCHUNK)
            def _next():
                drain_out(g, b)
                start_chunk(g + NB, b)
        return carry

    lax.fori_loop(0, NCHUNK // NB, super_body, 0)

    for b in range(NB):
        drain_out(NCHUNK - NB + b, b)


def kernel(x, min1_w, min2_w):
    xi = x.astype(jnp.int32)
    x0 = xi[:, :, 0].reshape(XR, D)
    x1 = xi[:, :, 1].reshape(XR, D)
    c, t = _idx_kernel(x0, x1, min1_w, min2_w)
    out = _sc_embed(c, t)
    return out.reshape(B, L, D)


# speedup vs baseline: 8.2952x; 1.1957x over previous
import functools
import jax
import jax.numpy as jnp
from jax import lax
from jax.experimental import pallas as pl
from jax.experimental.pallas import tpu as pltpu
from jax.experimental.pallas import tpu_sc as plsc

D = 128
B, L = 4096, 200
ROWS = B * L
XR = ROWS // D
NC, NS = 2, 16
NW = NC * NS
RPW = ROWS // NW
CH = 128
NCHUNK = RPW // CH
TROWS = 32
NB = 4

_mesh = plsc.VectorSubcoreMesh(core_axis_name="c", subcore_axis_name="s")


@functools.partial(
    pl.kernel,
    out_type=jax.ShapeDtypeStruct((ROWS, D), jnp.float32),
    mesh=_mesh,
    scratch_types=[
        pltpu.VMEM((8, D), jnp.float32),
        pltpu.VMEM((4, D), jnp.float32),
        pltpu.VMEM((TROWS, D), jnp.float32),
        pltpu.VMEM_SHARED((TROWS, D), jnp.float32),
        pltpu.VMEM((NCHUNK, CH), jnp.int32),
        pltpu.VMEM((NB, CH, D), jnp.float32),
        [pltpu.SemaphoreType.DMA] * NB,
        [pltpu.SemaphoreType.DMA] * NB,
    ],
)
def _sc_embed(c_hbm, min1_hbm, min2_hbm, out_hbm,
              m2_v, m1_v, t_v, t_sh, idx_v, rows_v, gsems, osems):
    cid = lax.axis_index("c")
    sid = lax.axis_index("s")
    wid = cid * NS + sid
    w0 = wid * RPW

    @pl.when(sid == 0)
    def _build():
        pltpu.sync_copy(min2_hbm, m2_v)
        pltpu.sync_copy(min1_hbm, m1_v)
        for a in range(8):
            for d in range(D // 16):
                v2 = m2_v[a, pl.ds(16 * d, 16)]
                for b in range(4):
                    t_v[4 * a + b, pl.ds(16 * d, 16)] = v2 + m1_v[b, pl.ds(16 * d, 16)]
        pltpu.sync_copy(t_v, t_sh)

    pltpu.sync_copy(c_hbm.at[pl.ds(wid * NCHUNK, NCHUNK)], idx_v)

    plsc.subcore_barrier()

    def start_chunk(g, b):
        pltpu.async_copy(t_sh.at[idx_v.at[g]], rows_v.at[b], gsems[b])

    def finish_chunk(g, b):
        base = w0 + g * CH
        pltpu.make_async_copy(t_sh.at[idx_v.at[g]], rows_v.at[b], gsems[b]).wait()
        pltpu.async_copy(rows_v.at[b], out_hbm.at[pl.ds(base, CH)], osems[b])

    def drain_out(g, b):
        base = w0 + g * CH
        pltpu.make_async_copy(
            rows_v.at[b], out_hbm.at[pl.ds(base, CH)], osems[b]).wait()

    for b in range(NB):
        start_chunk(b, b)

    def super_body(gg, carry):
        g0 = gg * NB
        for b in range(NB):
            g = g0 + b
            finish_chunk(g, b)

            @pl.when(g + NB < NCHUNK)
            def _next():
                drain_out(g, b)
                start_chunk(g + NB, b)
        return carry

    lax.fori_loop(0, NCHUNK // NB, super_body, 0)

    for b in range(NB):
        drain_out(NCHUNK - NB + b, b)


def kernel(x, min1_w, min2_w):
    xi = x.astype(jnp.int32)
    c = (xi[:, :, 0] * 4 + xi[:, :, 1]).reshape(XR, D)
    out = _sc_embed(c, min1_w, min2_w)
    return out.reshape(B, L, D)
